# Initial kernel scaffold; baseline (speedup 1.0000x reference)
import jax
import jax.numpy as jnp
from jax.experimental import pallas as pl


def kernel(x, tile_sigs, Ws, bs):
    raise NotImplementedError("write your pallas kernel here")


# speedup vs baseline: 3.0885x; 3.0885x over previous
import jax
import jax.numpy as jnp
from jax.experimental import pallas as pl

_BLK = 512


def _router_body(x_ref, sig_ref, w_ref, b_ref, o_ref):
    xb = x_ref[...]
    signs = jnp.sign(sig_ref[...])
    n, t_dim = xb.shape[0], signs.shape[0]
    scores = jax.lax.dot_general(xb, signs, (((1,), (1,)), ((), ())))
    m = jnp.max(scores, axis=1, keepdims=True)
    it = jax.lax.broadcasted_iota(jnp.int32, (n, t_dim), 1)
    idx = jnp.min(jnp.where(scores == m, it, t_dim), axis=1, keepdims=True)
    onehot = (it == idx).astype(jnp.float32)
    acc = jax.lax.dot_general(onehot, b_ref[...], (((1,), (0,)), ((), ())))
    for t in range(t_dim):
        xt = xb * onehot[:, t : t + 1]
        acc = acc + jax.lax.dot_general(xt, w_ref[t], (((1,), (1,)), ((), ())))
    o_ref[...] = acc


def kernel(x, tile_sigs, Ws, bs):
    b, s, d = x.shape
    t = tile_sigs.shape[0]
    x2 = x.reshape(s, d)
    out = pl.pallas_call(
        _router_body,
        grid=(s // _BLK,),
        in_specs=[
            pl.BlockSpec((_BLK, d), lambda i: (i, 0)),
            pl.BlockSpec((t, d), lambda i: (0, 0)),
            pl.BlockSpec((t, d, d), lambda i: (0, 0, 0)),
            pl.BlockSpec((t, d), lambda i: (0, 0)),
        ],
        out_specs=pl.BlockSpec((_BLK, d), lambda i: (i, 0)),
        out_shape=jax.ShapeDtypeStruct((s, d), jnp.float32),
    )(x2, tile_sigs, Ws, bs)
    return out.reshape(b, s, d)
